# Initial kernel scaffold; baseline (speedup 1.0000x reference)
import jax
import jax.numpy as jnp
from jax.experimental import pallas as pl


def kernel(output, target):
    raise NotImplementedError("write your pallas kernel here")


# speedup vs baseline: 1.9815x; 1.9815x over previous
import jax
import jax.numpy as jnp
from jax.experimental import pallas as pl
from jax.experimental.pallas import tpu as pltpu

_THRESHOLD = 0.7
_KEEP_NUM = 32768
_N = 131072
_C = 256

_BR = 2048
_NB = _N // _BR
_R2 = 1024
_C2 = _N // _R2

def _ce_body(x_ref, t_ref, loss_ref):
    x = x_ref[...]
    t = t_ref[0, 0, :]
    m = jnp.max(x, axis=1, keepdims=True)
    e = jnp.exp(x - m)
    s = jnp.sum(e, axis=1)
    logz = m[:, 0] + jnp.log(s)
    cols = jax.lax.broadcasted_iota(jnp.int32, (_BR, _C), 1)
    picked = jnp.sum(jnp.where(cols == t[:, None], x, 0.0), axis=1)
    loss_ref[0, 0, :] = logz - picked


def _select_body(loss_ref, out_ref, keys_ref):
    _SIGN = jnp.int32(-2 ** 31)
    x = loss_ref[...]
    bits = jax.lax.bitcast_convert_type(x, jnp.int32)
    ikey = jnp.where(bits >= 0, bits,
                     jnp.bitwise_xor(jnp.bitwise_not(bits), _SIGN))
    keys_ref[...] = ikey

    k1 = jnp.int32(_KEEP_NUM)
    k2 = jnp.int32(_KEEP_NUM + 1)

    def body(b, carry):
        p1, p2 = carry
        bit = jnp.left_shift(jnp.int32(1), 31 - b)
        c1 = jnp.bitwise_or(p1, bit)
        c2 = jnp.bitwise_or(p2, bit)
        k = keys_ref[...]
        cnt1 = jnp.sum((k >= jnp.bitwise_xor(c1, _SIGN)).astype(jnp.int32))
        cnt2 = jnp.sum((k >= jnp.bitwise_xor(c2, _SIGN)).astype(jnp.int32))
        p1 = jnp.where(cnt1 >= k1, c1, p1)
        p2 = jnp.where(cnt2 >= k2, c2, p2)
        return p1, p2

    p1, p2 = jax.lax.fori_loop(0, 32, body, (jnp.int32(0), jnp.int32(0)))
    ikey1 = jnp.bitwise_xor(p1, _SIGN)
    ikey2 = jnp.bitwise_xor(p2, _SIGN)

    k = keys_ref[...]
    v1 = jnp.max(jnp.where(k == ikey1, x, -jnp.inf))
    v2 = jnp.max(jnp.where(k == ikey2, x, -jnp.inf))

    gt1 = k > ikey1
    cnt_top = jnp.sum(gt1.astype(jnp.float32))
    sum_top = jnp.sum(jnp.where(gt1, x, 0.0))
    branch_b = (sum_top + v1 * (jnp.float32(_KEEP_NUM) - cnt_top)) \
        / jnp.float32(_KEEP_NUM)

    m7 = x > jnp.float32(_THRESHOLD)
    sum7 = jnp.sum(jnp.where(m7, x, 0.0))
    cnt7 = jnp.maximum(jnp.sum(m7.astype(jnp.float32)), 1.0)
    branch_a = sum7 / cnt7

    res = jnp.where(v2 > jnp.float32(_THRESHOLD), branch_a, branch_b)
    out_ref[...] = jnp.broadcast_to(res, (1, 1))


def kernel(output, target):
    t3 = target.reshape(_NB, 1, _BR)
    loss = pl.pallas_call(
        _ce_body,
        grid=(_NB,),
        in_specs=[
            pl.BlockSpec((_BR, _C), lambda i: (i, 0)),
            pl.BlockSpec((1, 1, _BR), lambda i: (i, 0, 0)),
        ],
        out_specs=pl.BlockSpec((1, 1, _BR), lambda i: (i, 0, 0)),
        out_shape=jax.ShapeDtypeStruct((_NB, 1, _BR), jnp.float32),
    )(output, t3)

    loss2 = loss.reshape(_R2, _C2)
    res = pl.pallas_call(
        _select_body,
        out_shape=jax.ShapeDtypeStruct((1, 1), jnp.float32),
        scratch_shapes=[pltpu.VMEM((_R2, _C2), jnp.int32)],
    )(loss2)
    return res[0, 0]
